# baseline (speedup 1.0000x reference)
import jax
import jax.numpy as jnp
from jax import lax
from jax.experimental import pallas as pl
from jax.experimental.pallas import tpu as pltpu
from jax.experimental.pallas import tpu_sc as plsc

BATCH = 16384
NUM_FRAMES = 100000
D_BETAS = 16

_NC, _NS = 2, 16
_NW = _NC * _NS
_BPW = BATCH // _NW
_BLK = 128
_NBLK = _BPW // _BLK

_TABLES = (
    ("go", 3, 2, 21846, 16),
    ("bp", 63, 5, 33289, 21),
    ("tr", 3, 2, 21846, 16),
    ("ex", 10, 2, 52429, 19),
)
_IOTA16 = None


def _sc_body(*refs):
    (ids_hbm, betas_hbm, go_hbm, bp_hbm, tr_hbm, ex_hbm,
     betas_out, go_out, bp_out, tr_out, ex_out) = refs[:11]
    tbl_hbm = (go_hbm, bp_hbm, tr_hbm, ex_hbm)
    tbl_out = (go_out, bp_out, tr_out, ex_out)
    r = list(refs[11:])
    idxb = r.pop(0)
    rem_b = [r.pop(0) for _ in _TABLES]
    gidx_b = [[r.pop(0) for _ in range(t[2])] for t in _TABLES]
    stg_b = [r.pop(0) for _ in _TABLES]
    out_b = [r.pop(0) for _ in _TABLES]
    betas_row = r.pop(0)
    betas_stg = r.pop(0)
    sem = r.pop(0)

    wid = lax.axis_index("s") * _NC + lax.axis_index("c")
    base = wid * _BPW
    iota = lax.iota(jnp.int32, 16)

    pltpu.sync_copy(betas_hbm.at[0], betas_row)
    row = betas_row[...]

    def _fill(i, carry):
        betas_stg[i, :] = row
        return carry

    lax.fori_loop(0, _BLK, _fill, 0)

    for b in range(_NBLK):
        blk = base + b * _BLK
        pltpu.sync_copy(ids_hbm.at[pl.ds(blk, _BLK)], idxb)

        for g in range(_BLK // 16):
            sl = pl.ds(g * 16, 16)
            vid = idxb[sl]
            for t, (_, d, span, _, _) in enumerate(_TABLES):
                nrows = (NUM_FRAMES * d) // 16 - 1
                prod = vid * d
                gb = prod >> 4
                rem_b[t][sl] = prod & 15
                for s in range(span):
                    gidx_b[t][s][sl] = jnp.minimum(gb + s, nrows)

        copies = []
        for t, (_, d, span, _, _) in enumerate(_TABLES):
            for s in range(span):
                copies.append(pltpu.async_copy(
                    tbl_hbm[t].at[gidx_b[t][s]], stg_b[t].at[s], sem))
        for c in copies:
            c.wait()

        for t, (_, d, span, mm, ss) in enumerate(_TABLES):
            nchunk = (_BLK * d) // 16
            stg = stg_b[t]
            remb = rem_b[t]
            outb = out_b[t]

            def _repack(c, carry, d=d, mm=mm, ss=ss, stg=stg, remb=remb,
                        outb=outb):
                p = c * 16 + iota
                i = (p * mm) >> ss
                k = p - i * d
                rem = plsc.load_gather(remb, [i])
                rk = rem + k
                s = rk >> 4
                wl = rk & 15
                outb[pl.ds(c * 16, 16)] = plsc.load_gather(stg, [s, i, wl])
                return carry

            lax.fori_loop(0, nchunk, _repack, 0)

        for t, (_, d, _, _, _) in enumerate(_TABLES):
            pltpu.sync_copy(out_b[t], tbl_out[t].at[pl.ds(blk * d, _BLK * d)])
        pltpu.sync_copy(betas_stg, betas_out.at[pl.ds(blk, _BLK)])


@jax.jit
def kernel(frame_ids, betas_w, global_orient_w, body_pose_w, transl_w,
           expression_w):
    mesh = plsc.VectorSubcoreMesh(core_axis_name="c", subcore_axis_name="s")
    f32 = jnp.float32
    out_type = (
        jax.ShapeDtypeStruct((BATCH, D_BETAS), f32),
        jax.ShapeDtypeStruct((BATCH * 3,), f32),
        jax.ShapeDtypeStruct((BATCH * 63,), f32),
        jax.ShapeDtypeStruct((BATCH * 3,), f32),
        jax.ShapeDtypeStruct((BATCH * 10,), f32),
    )
    scratch = [pltpu.VMEM((_BLK,), jnp.int32)]
    scratch += [pltpu.VMEM((_BLK,), jnp.int32) for _ in _TABLES]
    for _, _, span, _, _ in _TABLES:
        scratch += [pltpu.VMEM((_BLK,), jnp.int32) for _ in range(span)]
    scratch += [pltpu.VMEM((span, _BLK, 16), f32)
                for _, _, span, _, _ in _TABLES]
    scratch += [pltpu.VMEM((_BLK * d,), f32) for _, d, _, _, _ in _TABLES]
    scratch += [pltpu.VMEM((16,), f32), pltpu.VMEM((_BLK, D_BETAS), f32),
                pltpu.SemaphoreType.DMA]
    run = pl.kernel(
        _sc_body,
        out_type=out_type,
        mesh=mesh,
        scratch_types=scratch,
        compiler_params=pltpu.CompilerParams(
            use_tc_tiling_on_sc=False, needs_layout_passes=False),
    )
    ids = frame_ids.astype(jnp.int32)
    betas, go, bp, tr, ex = run(
        ids, betas_w,
        global_orient_w.reshape(-1, 16), body_pose_w.reshape(-1, 16),
        transl_w.reshape(-1, 16), expression_w.reshape(-1, 16))
    return (betas, go.reshape(BATCH, 3), bp.reshape(BATCH, 63),
            tr.reshape(BATCH, 3), ex.reshape(BATCH, 10))


# speedup vs baseline: 1.0758x; 1.0758x over previous
import jax
import jax.numpy as jnp
from jax import lax
from jax.experimental import pallas as pl
from jax.experimental.pallas import tpu as pltpu
from jax.experimental.pallas import tpu_sc as plsc

BATCH = 16384
NUM_FRAMES = 100000
D_BETAS = 16

_NC, _NS = 2, 16
_NW = _NC * _NS
_BPW = BATCH // _NW
_BLK = 128
_NBLK = _BPW // _BLK

_TABLES = (
    ("go", 3, 2, 21846, 16),
    ("bp", 63, 5, 33289, 21),
    ("tr", 3, 2, 21846, 16),
    ("ex", 10, 2, 52429, 19),
)
_IOTA16 = None


def _sc_body(*refs):
    (ids_hbm, betas_hbm, go_hbm, bp_hbm, tr_hbm, ex_hbm,
     betas_out, go_out, bp_out, tr_out, ex_out) = refs[:11]
    tbl_hbm = (go_hbm, bp_hbm, tr_hbm, ex_hbm)
    tbl_out = (go_out, bp_out, tr_out, ex_out)
    r = list(refs[11:])
    idxb = r.pop(0)
    rem_b = [r.pop(0) for _ in _TABLES]
    gidx_b = [[r.pop(0) for _ in range(t[2])] for t in _TABLES]
    stg_b = [r.pop(0) for _ in _TABLES]
    out_b = [r.pop(0) for _ in _TABLES]
    betas_row = r.pop(0)
    betas_stg = r.pop(0)
    sem = r.pop(0)

    wid = lax.axis_index("s") * _NC + lax.axis_index("c")
    base = wid * _BPW
    iota = lax.iota(jnp.int32, 16)

    pltpu.sync_copy(betas_hbm.at[0], betas_row)
    row = betas_row[...]

    @plsc.parallel_loop(0, _BLK, unroll=8)
    def _fill(i):
        betas_stg[i, :] = row

    for b in range(_NBLK):
        blk = base + b * _BLK
        pltpu.sync_copy(ids_hbm.at[pl.ds(blk, _BLK)], idxb)

        for g in range(_BLK // 16):
            sl = pl.ds(g * 16, 16)
            vid = idxb[sl]
            for t, (_, d, span, _, _) in enumerate(_TABLES):
                nrows = (NUM_FRAMES * d) // 16 - 1
                prod = vid * d
                gb = prod >> 4
                rem_b[t][sl] = prod & 15
                for s in range(span):
                    gidx_b[t][s][sl] = jnp.minimum(gb + s, nrows)

        copies = []
        for t, (_, d, span, _, _) in enumerate(_TABLES):
            for s in range(span):
                copies.append(pltpu.async_copy(
                    tbl_hbm[t].at[gidx_b[t][s]], stg_b[t].at[s], sem))
        for c in copies:
            c.wait()

        for t, (_, d, span, mm, ss) in enumerate(_TABLES):
            nchunk = (_BLK * d) // 16
            stg = stg_b[t]
            remb = rem_b[t]
            outb = out_b[t]

            @plsc.parallel_loop(0, nchunk, unroll=8)
            def _repack(c, d=d, mm=mm, ss=ss, stg=stg, remb=remb, outb=outb):
                p = c * 16 + iota
                i = (p * mm) >> ss
                k = p - i * d
                rem = plsc.load_gather(remb, [i])
                rk = rem + k
                s = rk >> 4
                wl = rk & 15
                outb[pl.ds(c * 16, 16)] = plsc.load_gather(stg, [s, i, wl])

        for t, (_, d, _, _, _) in enumerate(_TABLES):
            pltpu.sync_copy(out_b[t], tbl_out[t].at[pl.ds(blk * d, _BLK * d)])
        pltpu.sync_copy(betas_stg, betas_out.at[pl.ds(blk, _BLK)])


@jax.jit
def kernel(frame_ids, betas_w, global_orient_w, body_pose_w, transl_w,
           expression_w):
    mesh = plsc.VectorSubcoreMesh(core_axis_name="c", subcore_axis_name="s")
    f32 = jnp.float32
    out_type = (
        jax.ShapeDtypeStruct((BATCH, D_BETAS), f32),
        jax.ShapeDtypeStruct((BATCH * 3,), f32),
        jax.ShapeDtypeStruct((BATCH * 63,), f32),
        jax.ShapeDtypeStruct((BATCH * 3,), f32),
        jax.ShapeDtypeStruct((BATCH * 10,), f32),
    )
    scratch = [pltpu.VMEM((_BLK,), jnp.int32)]
    scratch += [pltpu.VMEM((_BLK,), jnp.int32) for _ in _TABLES]
    for _, _, span, _, _ in _TABLES:
        scratch += [pltpu.VMEM((_BLK,), jnp.int32) for _ in range(span)]
    scratch += [pltpu.VMEM((span, _BLK, 16), f32)
                for _, _, span, _, _ in _TABLES]
    scratch += [pltpu.VMEM((_BLK * d,), f32) for _, d, _, _, _ in _TABLES]
    scratch += [pltpu.VMEM((16,), f32), pltpu.VMEM((_BLK, D_BETAS), f32),
                pltpu.SemaphoreType.DMA]
    run = pl.kernel(
        _sc_body,
        out_type=out_type,
        mesh=mesh,
        scratch_types=scratch,
        compiler_params=pltpu.CompilerParams(
            use_tc_tiling_on_sc=False, needs_layout_passes=False),
    )
    ids = frame_ids.astype(jnp.int32)
    betas, go, bp, tr, ex = run(
        ids, betas_w,
        global_orient_w.reshape(-1, 16), body_pose_w.reshape(-1, 16),
        transl_w.reshape(-1, 16), expression_w.reshape(-1, 16))
    return (betas, go.reshape(BATCH, 3), bp.reshape(BATCH, 63),
            tr.reshape(BATCH, 3), ex.reshape(BATCH, 10))


# speedup vs baseline: 1.0766x; 1.0008x over previous
import jax
import jax.numpy as jnp
from jax import lax
from jax.experimental import pallas as pl
from jax.experimental.pallas import tpu as pltpu
from jax.experimental.pallas import tpu_sc as plsc

BATCH = 16384
NUM_FRAMES = 100000
D_BETAS = 16

_NC, _NS = 2, 16
_NW = _NC * _NS
_BPW = BATCH // _NW
_BLK = 128
_NBLK = _BPW // _BLK

_TABLES = (
    ("go", 3, 2, 21846, 16),
    ("bp", 63, 5, 33289, 21),
    ("tr", 3, 2, 21846, 16),
    ("ex", 10, 2, 52429, 19),
)
_IOTA16 = None


def _sc_body(*refs):
    (ids_hbm, betas_hbm, go_hbm, bp_hbm, tr_hbm, ex_hbm,
     betas_out, go_out, bp_out, tr_out, ex_out) = refs[:11]
    tbl_hbm = (go_hbm, bp_hbm, tr_hbm, ex_hbm)
    tbl_out = (go_out, bp_out, tr_out, ex_out)
    r = list(refs[11:])
    idxb = r.pop(0)
    rem_b = [r.pop(0) for _ in _TABLES]
    gidx_b = [[r.pop(0) for _ in range(t[2])] for t in _TABLES]
    stg_b = [r.pop(0) for _ in _TABLES]
    out_b = [r.pop(0) for _ in _TABLES]
    betas_row = r.pop(0)
    betas_stg = r.pop(0)
    sem = r.pop(0)

    wid = lax.axis_index("s") * _NC + lax.axis_index("c")
    base = wid * _BPW
    iota = lax.iota(jnp.int32, 16)

    pltpu.sync_copy(betas_hbm.at[0], betas_row)
    row = betas_row[...]

    @plsc.parallel_loop(0, _BLK, unroll=8)
    def _fill(i):
        betas_stg[i, :] = row

    for b in range(_NBLK):
        blk = base + b * _BLK
        with jax.named_scope("idx_load"):
            pltpu.sync_copy(ids_hbm.at[pl.ds(blk, _BLK)], idxb)

        for g in range(_BLK // 16):
            sl = pl.ds(g * 16, 16)
            vid = idxb[sl]
            for t, (_, d, span, _, _) in enumerate(_TABLES):
                nrows = (NUM_FRAMES * d) // 16 - 1
                prod = vid * d
                gb = prod >> 4
                rem_b[t][sl] = prod & 15
                for s in range(span):
                    gidx_b[t][s][sl] = jnp.minimum(gb + s, nrows)

        with jax.named_scope("gather_fire"):
            copies = []
            for t, (_, d, span, _, _) in enumerate(_TABLES):
                for s in range(span):
                    copies.append(pltpu.async_copy(
                        tbl_hbm[t].at[gidx_b[t][s]], stg_b[t].at[s], sem))
        with jax.named_scope("gather_wait"):
            for c in copies:
                c.wait()

        with jax.named_scope("repack"):
            for t, (_, d, span, mm, ss) in enumerate(_TABLES):
                nchunk = (_BLK * d) // 16
                stg = stg_b[t]
                remb = rem_b[t]
                outb = out_b[t]

                @plsc.parallel_loop(0, nchunk, unroll=8)
                def _repack(c, d=d, mm=mm, ss=ss, stg=stg, remb=remb,
                            outb=outb):
                    p = c * 16 + iota
                    i = (p * mm) >> ss
                    k = p - i * d
                    rem = plsc.load_gather(remb, [i])
                    rk = rem + k
                    s = rk >> 4
                    wl = rk & 15
                    outb[pl.ds(c * 16, 16)] = plsc.load_gather(stg, [s, i, wl])

        with jax.named_scope("out_copy"):
            for t, (_, d, _, _, _) in enumerate(_TABLES):
                pltpu.sync_copy(out_b[t],
                                tbl_out[t].at[pl.ds(blk * d, _BLK * d)])
            pltpu.sync_copy(betas_stg, betas_out.at[pl.ds(blk, _BLK)])


@jax.jit
def kernel(frame_ids, betas_w, global_orient_w, body_pose_w, transl_w,
           expression_w):
    mesh = plsc.VectorSubcoreMesh(core_axis_name="c", subcore_axis_name="s")
    f32 = jnp.float32
    out_type = (
        jax.ShapeDtypeStruct((BATCH, D_BETAS), f32),
        jax.ShapeDtypeStruct((BATCH * 3,), f32),
        jax.ShapeDtypeStruct((BATCH * 63,), f32),
        jax.ShapeDtypeStruct((BATCH * 3,), f32),
        jax.ShapeDtypeStruct((BATCH * 10,), f32),
    )
    scratch = [pltpu.VMEM((_BLK,), jnp.int32)]
    scratch += [pltpu.VMEM((_BLK,), jnp.int32) for _ in _TABLES]
    for _, _, span, _, _ in _TABLES:
        scratch += [pltpu.VMEM((_BLK,), jnp.int32) for _ in range(span)]
    scratch += [pltpu.VMEM((span, _BLK, 16), f32)
                for _, _, span, _, _ in _TABLES]
    scratch += [pltpu.VMEM((_BLK * d,), f32) for _, d, _, _, _ in _TABLES]
    scratch += [pltpu.VMEM((16,), f32), pltpu.VMEM((_BLK, D_BETAS), f32),
                pltpu.SemaphoreType.DMA]
    run = pl.kernel(
        _sc_body,
        out_type=out_type,
        mesh=mesh,
        scratch_types=scratch,
        compiler_params=pltpu.CompilerParams(
            use_tc_tiling_on_sc=False, needs_layout_passes=False),
    )
    ids = frame_ids.astype(jnp.int32)
    betas, go, bp, tr, ex = run(
        ids, betas_w,
        global_orient_w.reshape(-1, 16), body_pose_w.reshape(-1, 16),
        transl_w.reshape(-1, 16), expression_w.reshape(-1, 16))
    return (betas, go.reshape(BATCH, 3), bp.reshape(BATCH, 63),
            tr.reshape(BATCH, 3), ex.reshape(BATCH, 10))


# speedup vs baseline: 1.0946x; 1.0167x over previous
import jax
import jax.numpy as jnp
from jax import lax
from jax.experimental import pallas as pl
from jax.experimental.pallas import tpu as pltpu
from jax.experimental.pallas import tpu_sc as plsc

BATCH = 16384
NUM_FRAMES = 100000
D_BETAS = 16

_NC, _NS = 2, 16
_NW = _NC * _NS
_BPW = BATCH // _NW
_HALF = 256
_NH = _BPW // _HALF
_JB = _HALF // 128

_TABLES = (
    ("go", 3, 2, 21846, 16),
    ("bp", 63, 5, 33289, 21),
    ("tr", 3, 2, 21846, 16),
    ("ex", 10, 2, 52429, 19),
)


def _compute_indices(idx2, rem_b, gidx_b, iota16):
    del iota16
    for j in range(_JB):
        for g in range(128 // 16):
            sl = pl.ds(g * 16, 16)
            vid = idx2[j, sl]
            fl = pl.ds(j * 128 + g * 16, 16)
            for t, (_, d, span, _, _) in enumerate(_TABLES):
                nrows = (NUM_FRAMES * d) // 16 - 1
                prod = vid * d
                gb = prod >> 4
                rem_b[t][fl] = prod & 15
                for s in range(span):
                    gidx_b[t][s][fl] = jnp.minimum(gb + s, nrows)


def _sc_body(*refs):
    (ids_hbm, betas_hbm, go_hbm, bp_hbm, tr_hbm, ex_hbm,
     betas_out, go_out, bp_out, tr_out, ex_out) = refs[:11]
    tbl_hbm = (go_hbm, bp_hbm, tr_hbm, ex_hbm)
    tbl_out = (go_out, bp_out, tr_out, ex_out)
    r = list(refs[11:])
    idx2 = [r.pop(0) for _ in range(_NH)]
    rem_b = [[r.pop(0) for _ in _TABLES] for _ in range(_NH)]
    gidx_b = [[[r.pop(0) for _ in range(t[2])] for t in _TABLES]
              for _ in range(_NH)]
    stg_b = [[r.pop(0) for _ in _TABLES] for _ in range(_NH)]
    out_b = [r.pop(0) for _ in _TABLES]
    betas_row = r.pop(0)
    betas_stg = r.pop(0)
    sems = [r.pop(0) for _ in range(_NH)]

    wid = lax.axis_index("s") * _NC + lax.axis_index("c")
    base = wid * _BPW
    iota = lax.iota(jnp.int32, 16)

    def load_and_fire(h):
        hb = base + h * _HALF
        for j in range(_JB):
            pltpu.sync_copy(ids_hbm.at[pl.ds(hb + j * 128, 128)],
                            idx2[h].at[j])
        _compute_indices(idx2[h], rem_b[h], gidx_b[h], iota)
        copies = []
        for t, (_, d, span, _, _) in enumerate(_TABLES):
            for s in range(span):
                copies.append(pltpu.async_copy(
                    tbl_hbm[t].at[gidx_b[h][t][s]], stg_b[h][t].at[s],
                    sems[h]))
        return copies

    def repack_and_store(h):
        hb = base + h * _HALF
        for t, (_, d, span, mm, ss) in enumerate(_TABLES):
            nchunk = (_HALF * d) // 16
            stg = stg_b[h][t]
            remb = rem_b[h][t]
            outb = out_b[t]

            @plsc.parallel_loop(0, nchunk, unroll=8)
            def _repack(c, d=d, mm=mm, ss=ss, stg=stg, remb=remb, outb=outb):
                p = c * 16 + iota
                i = (p * mm) >> ss
                k = p - i * d
                rem = plsc.load_gather(remb, [i])
                rk = rem + k
                s = rk >> 4
                wl = rk & 15
                outb[pl.ds(c * 16, 16)] = plsc.load_gather(stg, [s, i, wl])

        for t, (_, d, _, _, _) in enumerate(_TABLES):
            pltpu.sync_copy(out_b[t], tbl_out[t].at[pl.ds(hb * d, _HALF * d)])

    with jax.named_scope("fire0"):
        c0 = load_and_fire(0)
    with jax.named_scope("fire1"):
        c1 = load_and_fire(1)

    with jax.named_scope("betas"):
        pltpu.sync_copy(betas_hbm.at[0], betas_row)
        row = betas_row[...]

        @plsc.parallel_loop(0, _HALF, unroll=8)
        def _fill(i):
            betas_stg[i, :] = row

        for h in range(_NH):
            pltpu.sync_copy(betas_stg,
                            betas_out.at[pl.ds(base + h * _HALF, _HALF)])

    with jax.named_scope("wait0"):
        for c in c0:
            c.wait()
    with jax.named_scope("repack0"):
        repack_and_store(0)
    with jax.named_scope("wait1"):
        for c in c1:
            c.wait()
    with jax.named_scope("repack1"):
        repack_and_store(1)


@jax.jit
def kernel(frame_ids, betas_w, global_orient_w, body_pose_w, transl_w,
           expression_w):
    mesh = plsc.VectorSubcoreMesh(core_axis_name="c", subcore_axis_name="s")
    f32 = jnp.float32
    i32 = jnp.int32
    out_type = (
        jax.ShapeDtypeStruct((BATCH, D_BETAS), f32),
        jax.ShapeDtypeStruct((BATCH * 3,), f32),
        jax.ShapeDtypeStruct((BATCH * 63,), f32),
        jax.ShapeDtypeStruct((BATCH * 3,), f32),
        jax.ShapeDtypeStruct((BATCH * 10,), f32),
    )
    scratch = []
    scratch += [pltpu.VMEM((_JB, 128), i32) for _ in range(_NH)]
    for _ in range(_NH):
        scratch += [pltpu.VMEM((_HALF,), i32) for _ in _TABLES]
    for _ in range(_NH):
        for _, _, span, _, _ in _TABLES:
            scratch += [pltpu.VMEM((_HALF,), i32) for _ in range(span)]
    for _ in range(_NH):
        scratch += [pltpu.VMEM((span, _HALF, 16), f32)
                    for _, _, span, _, _ in _TABLES]
    scratch += [pltpu.VMEM((_HALF * d,), f32) for _, d, _, _, _ in _TABLES]
    scratch += [pltpu.VMEM((16,), f32), pltpu.VMEM((_HALF, D_BETAS), f32)]
    scratch += [pltpu.SemaphoreType.DMA for _ in range(_NH)]
    run = pl.kernel(
        _sc_body,
        out_type=out_type,
        mesh=mesh,
        scratch_types=scratch,
        compiler_params=pltpu.CompilerParams(
            use_tc_tiling_on_sc=False, needs_layout_passes=False),
    )
    ids = frame_ids.astype(i32)
    betas, go, bp, tr, ex = run(
        ids, betas_w,
        global_orient_w.reshape(-1, 16), body_pose_w.reshape(-1, 16),
        transl_w.reshape(-1, 16), expression_w.reshape(-1, 16))
    return (betas, go.reshape(BATCH, 3), bp.reshape(BATCH, 63),
            tr.reshape(BATCH, 3), ex.reshape(BATCH, 10))
